# baseline (speedup 1.0000x reference)
import functools

import jax
import jax.numpy as jnp
from jax.experimental import pallas as pl

MAX_DEPTH = 8
_BLK = 1024


def _mlp_body(n_in, *refs):
    b1 = refs[2 * n_in]
    w2 = refs[2 * n_in + 1]
    b2 = refs[2 * n_in + 2]
    out = refs[-1]
    h = b1[...].astype(jnp.float32)
    for i in range(n_in):
        xi = refs[i][...]
        wi = refs[n_in + i][...]
        if xi.shape[1] == 1:
            h = h + xi.astype(jnp.float32) * wi[...].astype(jnp.float32)
        else:
            h = h + jax.lax.dot_general(
                xi.astype(jnp.bfloat16), wi.astype(jnp.bfloat16),
                (((1,), (0,)), ((), ())), preferred_element_type=jnp.float32)
    h = jnp.maximum(h, 0.0)
    y = jax.lax.dot_general(
        h.astype(jnp.bfloat16), w2[...].astype(jnp.bfloat16),
        (((1,), (0,)), ((), ())), preferred_element_type=jnp.float32)
    out[...] = y + b2[...].astype(jnp.float32)


def _mlp(parts, b1, w2, b2):
    n_in = len(parts)
    r = parts[0][0].shape[0]
    o = w2.shape[1]
    blk = min(_BLK, r)
    grid = (pl.cdiv(r, blk),)
    in_specs = (
        [pl.BlockSpec((blk, xi.shape[1]), lambda i: (i, 0)) for xi, _ in parts]
        + [pl.BlockSpec(wi.shape, lambda i: (0, 0)) for _, wi in parts]
        + [pl.BlockSpec((1, b1.shape[1]), lambda i: (0, 0)),
           pl.BlockSpec(w2.shape, lambda i: (0, 0)),
           pl.BlockSpec((1, b2.shape[1]), lambda i: (0, 0))]
    )
    fn = pl.pallas_call(
        functools.partial(_mlp_body, n_in),
        grid=grid,
        in_specs=in_specs,
        out_specs=pl.BlockSpec((blk, o), lambda i: (i, 0)),
        out_shape=jax.ShapeDtypeStruct((r, o), jnp.float32),
    )
    args = [xi for xi, _ in parts] + [wi for _, wi in parts] + [b1, w2, b2]
    return fn(*args)


def _mlp1(x, p):
    return _mlp([(x, p["W1"])], p["b1"].reshape(1, -1), p["W2"],
                p["b2"].reshape(1, -1))


def kernel(x, parent_edge_features, parent_light_edge_features, params,
           edge_index, depths, states):
    pef = _mlp1(parent_edge_features, params["edge"])
    plef = _mlp1(parent_light_edge_features, params["edge"])
    x = _mlp1(x, params["enc"])
    n = x.shape[0]
    parents = jnp.zeros((n,), dtype=edge_index.dtype).at[edge_index[0]].set(
        edge_index[1])

    pm = params["merger"]
    w1m_l, w1m_r, w1m_p = (pm["W1"][:128], pm["W1"][128:256], pm["W1"][256:])
    plp = params["lep"]
    w1lp_x, w1lp_e = plp["W1"][:128], plp["W1"][128:]
    plm = params["lem"]
    w1lm_x, w1lm_m = plm["W1"][:128], plm["W1"][128:]

    for depth in range(MAX_DEPTH, 0, -1):
        mask_depth = depths == depth
        lm = (mask_depth & (states == 0))[:, None].astype(x.dtype)
        rm = (mask_depth & (states == 1))[:, None].astype(x.dtype)
        left = jnp.zeros_like(x).at[parents].add(x * lm)
        right = jnp.zeros_like(x).at[parents].add(x * rm)
        x_parents = _mlp([(left, w1m_l), (right, w1m_r), (pef, w1m_p)],
                         pm["b1"].reshape(1, -1), pm["W2"],
                         pm["b2"].reshape(1, -1))
        parents_mask = jnp.zeros((n, 1), x.dtype).at[parents].add(lm) != 0
        hm = (mask_depth & (states == 3))[:, None].astype(x.dtype)
        processed = _mlp([(x, w1lp_x), (plef, w1lp_e)],
                         plp["b1"].reshape(1, -1), plp["W2"],
                         plp["b2"].reshape(1, -1))
        merged_heads_sum = jnp.zeros_like(x).at[parents].add(processed * hm)
        designated = jnp.zeros((n, 1), x.dtype).at[parents].add(hm) != 0
        x_desig = jnp.where(designated, x, jnp.zeros_like(x))
        x_merged = _mlp([(x_desig, w1lm_x), (merged_heads_sum, w1lm_m)],
                        plm["b1"].reshape(1, -1), plm["W2"],
                        plm["b2"].reshape(1, -1))
        x = jnp.where(parents_mask, x_parents, x)
        x = jnp.where(designated, x_merged, x)

    max_depth = jnp.max(depths)
    states_f = states.astype(x.dtype)[:, None]
    pp = params["proc"]
    w1p_x, w1p_p, w1p_s = pp["W1"][:128], pp["W1"][128:256], pp["W1"][256:]
    for depth in range(1, MAX_DEPTH):
        mask = ((depths == depth) & (depth < max_depth))[:, None]
        merged = _mlp([(x, w1p_x), (x[parents], w1p_p), (states_f, w1p_s)],
                      pp["b1"].reshape(1, -1), pp["W2"],
                      pp["b2"].reshape(1, -1))
        x = jnp.where(mask, merged, x)

    return _mlp1(x, params["dec"])


# speedup vs baseline: 1.2392x; 1.2392x over previous
import functools

import jax
import jax.numpy as jnp
from jax import lax
from jax.experimental import pallas as pl
from jax.experimental.pallas import tpu as pltpu
from jax.experimental.pallas import tpu_sc as plsc

MAX_DEPTH = 8
_BLK = 1024

_CAPC = 8192
_CAPP = 4096
_CAPN = 16384
_PBITS = 17


def _preprocess(parents, depths, states, n):
    pmask = (1 << _PBITS) - 1
    role = jnp.where(states == 0, 0, jnp.where(states == 1, 1,
                     jnp.where(states == 3, 2, 3)))
    key = ((role * MAX_DEPTH + depths) << _PBITS) + parents
    order = jnp.argsort(key)
    skey = key[order]
    sdep = depths[order]
    srole = role[order]
    dpkey = skey & ((MAX_DEPTH << _PBITS) - 1)

    flag = jnp.concatenate([jnp.ones((1,), jnp.int32),
                            (skey[1:] != skey[:-1]).astype(jnp.int32)])
    cs = jnp.cumsum(flag)

    bounds = (jnp.arange(3)[:, None] * MAX_DEPTH +
              jnp.arange(MAX_DEPTH + 1)[None, :]) << _PBITS
    B = jnp.searchsorted(skey, bounds.reshape(-1)).reshape(3, MAX_DEPTH + 1)
    cs_pad = jnp.concatenate([cs, cs[-1:]])
    base_cs = cs_pad[jnp.clip(B, 0, n - 1)]

    elem_base = base_cs[jnp.clip(srole, 0, 2), sdep]
    slot_all = jnp.clip(cs - elem_base, 0, _CAPP - 1)

    d_rows = jnp.arange(1, MAX_DEPTH)

    def child_arrays(r):
        b0 = B[r, d_rows]
        b1 = B[r, d_rows + 1]
        pos = b0[:, None] + jnp.arange(_CAPC)[None, :]
        valid = pos < b1[:, None]
        posc = jnp.clip(pos, 0, n - 1)
        idx = jnp.where(valid, order[posc], n)
        slot = jnp.where(valid, slot_all[posc], _CAPP - 1)
        return idx.astype(jnp.int32), slot.astype(jnp.int32)

    l_idx, l_slot = child_arrays(0)
    h_idx, h_slot = child_arrays(2)

    def seg_table(r):
        sent = (((jnp.arange(7 * _CAPP) // _CAPP + 1) << _PBITS) + pmask
                ).astype(jnp.int32)
        in_reg = (srole == r) & (sdep >= 1) & (flag == 1)
        tgt = jnp.where(in_reg, (sdep - 1) * _CAPP + slot_all, 7 * _CAPP)
        return jnp.concatenate([sent, jnp.full((1,), 2**30, jnp.int32)]
                               ).at[tgt].set(dpkey.astype(jnp.int32))[:-1]

    up_keys = seg_table(0)
    hp_keys = seg_table(2)

    rpos = B[1, d_rows][:, None] + jnp.arange(_CAPC)[None, :]
    rvalid = rpos < B[1, d_rows + 1][:, None]
    rposc = jnp.clip(rpos, 0, n - 1)
    r_idx = jnp.where(rvalid, order[rposc], n).astype(jnp.int32)
    rkey = dpkey[rposc].astype(jnp.int32)
    loc = jnp.searchsorted(up_keys, rkey)
    locc = jnp.clip(loc, 0, 7 * _CAPP - 1)
    found = rvalid & (up_keys[locc] == rkey)
    r_slot = jnp.where(found, locc % _CAPP, _CAPP - 1).astype(jnp.int32)

    def slot_parent(keys):
        par = keys & pmask
        vald = par != pmask
        return par, vald

    up_par, up_valid = slot_parent(up_keys)
    hp_par, hp_valid = slot_parent(hp_keys)
    loc2 = jnp.clip(jnp.searchsorted(hp_keys, up_keys), 0, 7 * _CAPP - 1)
    up_in_hp = hp_keys[loc2] == up_keys
    up_tgt = jnp.where(up_valid & ~up_in_hp, up_par, n
                       ).reshape(7, _CAPP).astype(jnp.int32)
    hp_tgt = jnp.where(hp_valid, hp_par, n).reshape(7, _CAPP).astype(jnp.int32)
    hp_gidx = jnp.where(hp_valid, hp_par, n).reshape(7, _CAPP).astype(jnp.int32)
    up_gidx = jnp.where(up_valid, up_par, 0).reshape(7, _CAPP).astype(jnp.int32)

    h_idx_cl = jnp.minimum(h_idx, n - 1)

    rorder = jnp.argsort(depths).astype(jnp.int32)
    rB = jnp.searchsorted(depths[rorder], jnp.arange(MAX_DEPTH + 1))
    npos = rB[d_rows][:, None] + jnp.arange(_CAPN)[None, :]
    nvalid = npos < rB[d_rows + 1][:, None]
    nposc = jnp.clip(npos, 0, n - 1)
    ro_node = jnp.where(nvalid, rorder[nposc], n).astype(jnp.int32)
    ro_par = jnp.where(nvalid, parents[rorder[nposc]], n).astype(jnp.int32)
    ro_sf = jnp.where(nvalid, states[rorder[nposc]], 0).astype(jnp.float32)
    max_depth = jnp.max(depths)
    ro_tgt = jnp.where(nvalid & (d_rows[:, None] < max_depth), ro_node, n
                       ).astype(jnp.int32)

    return dict(l_idx=l_idx, l_slot=l_slot, r_idx=r_idx, r_slot=r_slot,
                h_idx=h_idx, h_slot=h_slot, h_idx_cl=h_idx_cl,
                up_tgt=up_tgt, hp_tgt=hp_tgt, hp_gidx=hp_gidx,
                up_gidx=up_gidx, ro_node=ro_node, ro_par=ro_par,
                ro_sf=ro_sf[..., None], ro_tgt=ro_tgt)


def _sds(shape, dtype=jnp.float32):
    return jax.ShapeDtypeStruct(shape, dtype)


def _mesh():
    return plsc.VectorSubcoreMesh(core_axis_name="c", subcore_axis_name="s")


def _wid():
    return lax.axis_index("s") * 2 + lax.axis_index("c")


@functools.lru_cache(maxsize=None)
def _sc_gather_h(np_rows, n):
    @functools.partial(
        pl.kernel, mesh=_mesh(),
        out_type=(_sds((_CAPC, 128)), _sds((_CAPC, 128))),
        scratch_types=[pltpu.VMEM((2, 128), jnp.int32),
                       pltpu.VMEM((128, 128), jnp.float32),
                       pltpu.SemaphoreType.DMA])
    def k(xb, plr, idx, xg, plg, idx_v, rows_v, sem):
        w = _wid()
        pltpu.sync_copy(idx.at[w], idx_v)
        for j in range(2):
            pltpu.async_copy(xb.at[idx_v.at[j]], rows_v, sem).wait()
            pltpu.sync_copy(rows_v, xg.at[pl.ds((w * 2 + j) * 128, 128)])
            pltpu.async_copy(plr.at[idx_v.at[j]], rows_v, sem).wait()
            pltpu.sync_copy(rows_v, plg.at[pl.ds((w * 2 + j) * 128, 128)])
    return k


@functools.lru_cache(maxsize=None)
def _sc_gather_ro(np_rows):
    @functools.partial(
        pl.kernel, mesh=_mesh(),
        out_type=(_sds((_CAPN, 128)), _sds((_CAPN, 128))),
        scratch_types=[pltpu.VMEM((4, 128), jnp.int32),
                       pltpu.VMEM((128, 128), jnp.float32),
                       pltpu.SemaphoreType.DMA])
    def k(xb, idx1, idx2, xn, xp, idx_v, rows_v, sem):
        w = _wid()
        pltpu.sync_copy(idx1.at[w], idx_v)
        for j in range(4):
            pltpu.async_copy(xb.at[idx_v.at[j]], rows_v, sem).wait()
            pltpu.sync_copy(rows_v, xn.at[pl.ds((w * 4 + j) * 128, 128)])
        pltpu.sync_copy(idx2.at[w], idx_v)
        for j in range(4):
            pltpu.async_copy(xb.at[idx_v.at[j]], rows_v, sem).wait()
            pltpu.sync_copy(rows_v, xp.at[pl.ds((w * 4 + j) * 128, 128)])
    return k


@functools.lru_cache(maxsize=None)
def _sc_accum(np_rows, n):
    @functools.partial(
        pl.kernel, mesh=_mesh(),
        out_type=(_sds((_CAPP, 128)), _sds((_CAPP, 128)), _sds((_CAPP, 128)),
                  _sds((_CAPP, 128)), _sds((_CAPP, 128))),
        scratch_types=[pltpu.VMEM_SHARED((_CAPP, 128), jnp.float32),
                       pltpu.VMEM_SHARED((_CAPP, 128), jnp.float32),
                       pltpu.VMEM_SHARED((_CAPP, 128), jnp.float32),
                       pltpu.VMEM((4, 128), jnp.int32),
                       pltpu.VMEM((2, 128), jnp.int32),
                       pltpu.VMEM((4, 128), jnp.int32),
                       pltpu.VMEM((128, 128), jnp.float32),
                       pltpu.SemaphoreType.DMA])
    def k(xb, proc, pefr, lidx, lslot, ridx, rslot, hslot, hpg, upg,
          leftsum, rightsum, mhs, xd, pefu,
          accl, accr, acch, idx_v, idx2_v, slot_v, rows_v, sem):
        c = lax.axis_index("c")
        s = lax.axis_index("s")

        def zero_row(i, carry):
            for q in range(8):
                rows_v[i, pl.ds(q * 16, 16)] = jnp.zeros((16,), jnp.float32)
            return carry
        lax.fori_loop(0, 128, zero_row, 0)

        @pl.when(c == 0)
        def _():
            for q in range(2):
                pltpu.sync_copy(rows_v, accl.at[pl.ds(s * 256 + q * 128, 128)])
                pltpu.sync_copy(rows_v, accr.at[pl.ds(s * 256 + q * 128, 128)])

        @pl.when(c == 1)
        def _():
            for q in range(2):
                pltpu.sync_copy(rows_v, acch.at[pl.ds(s * 256 + q * 128, 128)])

        plsc.subcore_barrier()

        @pl.when(c == 0)
        def _():
            pltpu.sync_copy(lidx.at[s], idx_v)
            pltpu.sync_copy(lslot.at[s], slot_v)
            for j in range(4):
                pltpu.async_copy(xb.at[idx_v.at[j]], rows_v, sem).wait()
                pltpu.sync_copy(rows_v, accl.at[slot_v.at[j]], add=True)
            pltpu.sync_copy(ridx.at[s], idx_v)
            pltpu.sync_copy(rslot.at[s], slot_v)
            for j in range(4):
                pltpu.async_copy(xb.at[idx_v.at[j]], rows_v, sem).wait()
                pltpu.sync_copy(rows_v, accr.at[slot_v.at[j]], add=True)

        @pl.when(c == 1)
        def _():
            pltpu.sync_copy(hslot.at[s], slot_v)
            for j in range(4):
                pltpu.sync_copy(proc.at[pl.ds(s * 512 + j * 128, 128)], rows_v)
                pltpu.sync_copy(rows_v, acch.at[slot_v.at[j]], add=True)
            pltpu.sync_copy(hpg.at[s], idx2_v)
            for j in range(2):
                pltpu.async_copy(xb.at[idx2_v.at[j]], rows_v, sem).wait()
                pltpu.sync_copy(rows_v, xd.at[pl.ds(s * 256 + j * 128, 128)])
            pltpu.sync_copy(upg.at[s], idx2_v)
            for j in range(2):
                pltpu.async_copy(pefr.at[idx2_v.at[j]], rows_v, sem).wait()
                pltpu.sync_copy(rows_v, pefu.at[pl.ds(s * 256 + j * 128, 128)])

        plsc.subcore_barrier()

        @pl.when(c == 0)
        def _():
            for q in range(2):
                sl = pl.ds(s * 256 + q * 128, 128)
                pltpu.sync_copy(accl.at[sl], leftsum.at[sl])
                pltpu.sync_copy(accr.at[sl], rightsum.at[sl])

        @pl.when(c == 1)
        def _():
            for q in range(2):
                sl = pl.ds(s * 256 + q * 128, 128)
                pltpu.sync_copy(acch.at[sl], mhs.at[sl])
    return k


@functools.lru_cache(maxsize=None)
def _sc_scatter(np_rows, nk):
    @functools.partial(
        pl.kernel, mesh=_mesh(), out_type=(),
        scratch_types=[pltpu.VMEM((nk, 128), jnp.int32),
                       pltpu.VMEM((128, 128), jnp.float32),
                       pltpu.SemaphoreType.DMA])
    def k(src, tgt, xb, idx_v, rows_v, sem):
        w = _wid()
        pltpu.sync_copy(tgt.at[w], idx_v)
        for j in range(nk):
            pltpu.sync_copy(src.at[pl.ds((w * nk + j) * 128, 128)], rows_v)
            pltpu.async_copy(rows_v, xb.at[idx_v.at[j]], sem).wait()
    return k


def _dot(a, b):
    return lax.dot_general(a.astype(jnp.bfloat16), b.astype(jnp.bfloat16),
                           (((1,), (0,)), ((), ())),
                           preferred_element_type=jnp.float32)


def _mlp_body(n_in, *refs):
    b1 = refs[2 * n_in]
    w2 = refs[2 * n_in + 1]
    b2 = refs[2 * n_in + 2]
    out = refs[-1]
    h = b1[...].astype(jnp.float32)
    for i in range(n_in):
        xi = refs[i][...]
        wi = refs[n_in + i][...]
        if xi.shape[1] == 1:
            h = h + xi.astype(jnp.float32) * wi[...].astype(jnp.float32)
        else:
            h = h + _dot(xi, wi)
    h = jnp.maximum(h, 0.0)
    out[...] = _dot(h, w2[...]) + b2[...].astype(jnp.float32)


def _mlp(parts, b1, w2, b2, out_rows=None):
    n_in = len(parts)
    r = out_rows if out_rows is not None else parts[0][0].shape[0]
    o = w2.shape[1]
    blk = min(_BLK, r)
    grid = (pl.cdiv(r, blk),)
    in_specs = (
        [pl.BlockSpec((blk, xi.shape[1]), lambda i: (i, 0)) for xi, _ in parts]
        + [pl.BlockSpec(wi.shape, lambda i: (0, 0)) for _, wi in parts]
        + [pl.BlockSpec((1, b1.shape[1]), lambda i: (0, 0)),
           pl.BlockSpec(w2.shape, lambda i: (0, 0)),
           pl.BlockSpec((1, b2.shape[1]), lambda i: (0, 0))]
    )
    fn = pl.pallas_call(
        functools.partial(_mlp_body, n_in),
        grid=grid,
        in_specs=in_specs,
        out_specs=pl.BlockSpec((blk, o), lambda i: (i, 0)),
        out_shape=jax.ShapeDtypeStruct((r, o), jnp.float32),
    )
    args = [xi for xi, _ in parts] + [wi for _, wi in parts] + [b1, w2, b2]
    return fn(*args)


def _mlp1(x, p, out_rows=None):
    return _mlp([(x, p["W1"])], p["b1"].reshape(1, -1), p["W2"],
                p["b2"].reshape(1, -1), out_rows=out_rows)


def _pair_body(*refs):
    (ls, rs, pu, xd, mh,
     w1ml, w1mr, w1mp, b1m, w2m, b2m,
     w1lx, w1lm, b1l, w2l, b2l, oup, ohp) = refs
    h = jnp.maximum(_dot(ls[...], w1ml[...]) + _dot(rs[...], w1mr[...])
                    + _dot(pu[...], w1mp[...]) + b1m[...], 0.0)
    oup[...] = _dot(h, w2m[...]) + b2m[...]
    h2 = jnp.maximum(_dot(xd[...], w1lx[...]) + _dot(mh[...], w1lm[...])
                     + b1l[...], 0.0)
    ohp[...] = _dot(h2, w2l[...]) + b2l[...]


def _pair_call(ls, rs, pu, xd, mh, pm, plm):
    blk = 1024
    args = [ls, rs, pu, xd, mh,
            pm["W1"][:128], pm["W1"][128:256], pm["W1"][256:],
            pm["b1"].reshape(1, -1), pm["W2"], pm["b2"].reshape(1, -1),
            plm["W1"][:128], plm["W1"][128:], plm["b1"].reshape(1, -1),
            plm["W2"], plm["b2"].reshape(1, -1)]
    in_specs = [pl.BlockSpec((blk, 128), lambda i: (i, 0))] * 5 + [
        pl.BlockSpec(a.shape, lambda i: (0, 0)) for a in args[5:]]
    return pl.pallas_call(
        _pair_body, grid=(_CAPP // blk,), in_specs=in_specs,
        out_specs=(pl.BlockSpec((blk, 128), lambda i: (i, 0)),
                   pl.BlockSpec((blk, 128), lambda i: (i, 0))),
        out_shape=(_sds((_CAPP, 128)), _sds((_CAPP, 128))))(*args)


def kernel(x, parent_edge_features, parent_light_edge_features, params,
           edge_index, depths, states):
    pef_raw = parent_edge_features
    plef_raw = parent_light_edge_features
    n = x.shape[0]
    np_rows = (n // _BLK + 1) * _BLK
    parents = jnp.zeros((n,), dtype=edge_index.dtype).at[edge_index[0]].set(
        edge_index[1])
    pre = _preprocess(parents, depths, states, n)

    x_enc = _mlp1(x, params["enc"], out_rows=np_rows)
    xref = jax.new_ref(x_enc)

    pe = params["edge"]
    pm = params["merger"]
    plp = params["lep"]
    plm = params["lem"]
    pp = params["proc"]
    pef = _mlp1(pef_raw, pe)
    plef = _mlp1(plef_raw, pe)
    w1lp_x, w1lp_e = plp["W1"][:128], plp["W1"][128:]

    gh = _sc_gather_h(np_rows, n)
    gro = _sc_gather_ro(np_rows)
    acc = _sc_accum(np_rows, n)
    sc1 = _sc_scatter(np_rows, 1)
    sc4 = _sc_scatter(np_rows, 4)

    for depth in range(MAX_DEPTH - 1, 0, -1):
        r = depth - 1
        xg, plg = gh(xref, plef,
                     pre["h_idx_cl"][r].reshape(32, 2, 128))
        processed = _mlp([(xg, w1lp_x), (plg, w1lp_e)],
                         plp["b1"].reshape(1, -1), plp["W2"],
                         plp["b2"].reshape(1, -1))
        ls, rs, mh, xd, pu = acc(
            xref, processed, pef,
            pre["l_idx"][r].reshape(16, 4, 128),
            pre["l_slot"][r].reshape(16, 4, 128),
            pre["r_idx"][r].reshape(16, 4, 128),
            pre["r_slot"][r].reshape(16, 4, 128),
            pre["h_slot"][r].reshape(16, 4, 128),
            pre["hp_gidx"][r].reshape(16, 2, 128),
            pre["up_gidx"][r].reshape(16, 2, 128))
        newx_up, newx_hp = _pair_call(ls, rs, pu, xd, mh, pm, plm)
        sc1(newx_up, pre["up_tgt"][r].reshape(32, 1, 128), xref)
        sc1(newx_hp, pre["hp_tgt"][r].reshape(32, 1, 128), xref)

    w1p_x, w1p_p, w1p_s = pp["W1"][:128], pp["W1"][128:256], pp["W1"][256:]
    for depth in range(1, MAX_DEPTH):
        r = depth - 1
        xn, xp = gro(xref,
                     pre["ro_node"][r].reshape(32, 4, 128),
                     pre["ro_par"][r].reshape(32, 4, 128))
        merged = _mlp([(xn, w1p_x), (xp, w1p_p), (pre["ro_sf"][r], w1p_s)],
                      pp["b1"].reshape(1, -1), pp["W2"],
                      pp["b2"].reshape(1, -1))
        sc4(merged, pre["ro_tgt"][r].reshape(32, 4, 128), xref)

    xfin = xref[...]
    return _mlp1(xfin, params["dec"])[:n]


# speedup vs baseline: 2.1838x; 1.7622x over previous
import functools

import jax
import jax.numpy as jnp
from jax import lax
from jax.experimental import pallas as pl
from jax.experimental.pallas import tpu as pltpu
from jax.experimental.pallas import tpu_sc as plsc

MAX_DEPTH = 8
_BLK = 1024

_CAPC = 4096
_CAPP = 4096
_CAPN = 16384
_PBITS = 17


def _preprocess(parents, depths, states, n):
    pmask = (1 << _PBITS) - 1
    role = jnp.where(states == 0, 0, jnp.where(states == 1, 1,
                     jnp.where(states == 3, 2, 3)))
    key = ((role * MAX_DEPTH + depths) << _PBITS) + parents
    order = jnp.argsort(key)
    skey = key[order]
    sdep = depths[order]
    srole = role[order]
    dpkey = skey & ((MAX_DEPTH << _PBITS) - 1)

    flag = jnp.concatenate([jnp.ones((1,), jnp.int32),
                            (skey[1:] != skey[:-1]).astype(jnp.int32)])
    cs = jnp.cumsum(flag)

    bounds = (jnp.arange(3)[:, None] * MAX_DEPTH +
              jnp.arange(MAX_DEPTH + 1)[None, :]) << _PBITS
    B = jnp.searchsorted(skey, bounds.reshape(-1)).reshape(3, MAX_DEPTH + 1)
    cs_pad = jnp.concatenate([cs, cs[-1:]])
    base_cs = cs_pad[jnp.clip(B, 0, n - 1)]

    elem_base = base_cs[jnp.clip(srole, 0, 2), sdep]
    slot_all = jnp.clip(cs - elem_base, 0, _CAPP - 1)

    d_rows = jnp.arange(1, MAX_DEPTH)

    def child_arrays(r):
        b0 = B[r, d_rows]
        b1 = B[r, d_rows + 1]
        pos = b0[:, None] + jnp.arange(_CAPC)[None, :]
        valid = pos < b1[:, None]
        posc = jnp.clip(pos, 0, n - 1)
        idx = jnp.where(valid, order[posc], n)
        slot = jnp.where(valid, slot_all[posc], _CAPP - 1)
        return idx.astype(jnp.int32), slot.astype(jnp.int32)

    l_idx, l_slot = child_arrays(0)
    h_idx, h_slot = child_arrays(2)

    def seg_table(r):
        sent = (((jnp.arange(7 * _CAPP) // _CAPP + 1) << _PBITS) + pmask
                ).astype(jnp.int32)
        in_reg = (srole == r) & (sdep >= 1) & (flag == 1)
        tgt = jnp.where(in_reg, (sdep - 1) * _CAPP + slot_all, 7 * _CAPP)
        return jnp.concatenate([sent, jnp.full((1,), 2**30, jnp.int32)]
                               ).at[tgt].set(dpkey.astype(jnp.int32))[:-1]

    up_keys = seg_table(0)
    hp_keys = seg_table(2)

    rpos = B[1, d_rows][:, None] + jnp.arange(_CAPC)[None, :]
    rvalid = rpos < B[1, d_rows + 1][:, None]
    rposc = jnp.clip(rpos, 0, n - 1)
    r_idx = jnp.where(rvalid, order[rposc], n).astype(jnp.int32)
    rkey = dpkey[rposc].astype(jnp.int32)
    loc = jnp.searchsorted(up_keys, rkey)
    locc = jnp.clip(loc, 0, 7 * _CAPP - 1)
    found = rvalid & (up_keys[locc] == rkey)
    r_slot = jnp.where(found, locc % _CAPP, _CAPP - 1).astype(jnp.int32)

    def slot_parent(keys):
        par = keys & pmask
        vald = par != pmask
        return par, vald

    up_par, up_valid = slot_parent(up_keys)
    hp_par, hp_valid = slot_parent(hp_keys)
    loc2 = jnp.clip(jnp.searchsorted(hp_keys, up_keys), 0, 7 * _CAPP - 1)
    up_in_hp = hp_keys[loc2] == up_keys
    up_tgt = jnp.where(up_valid & ~up_in_hp, up_par, n
                       ).reshape(7, _CAPP).astype(jnp.int32)
    hp_tgt = jnp.where(hp_valid, hp_par, n).reshape(7, _CAPP).astype(jnp.int32)
    hp_gidx = jnp.where(hp_valid, hp_par, n).reshape(7, _CAPP).astype(jnp.int32)
    up_gidx = jnp.where(up_valid, up_par, 0).reshape(7, _CAPP).astype(jnp.int32)

    h_idx_cl = jnp.minimum(h_idx, n - 1)

    rorder = jnp.argsort(depths).astype(jnp.int32)
    rB = jnp.searchsorted(depths[rorder], jnp.arange(MAX_DEPTH + 1))
    npos = rB[d_rows][:, None] + jnp.arange(_CAPN)[None, :]
    nvalid = npos < rB[d_rows + 1][:, None]
    nposc = jnp.clip(npos, 0, n - 1)
    ro_node = jnp.where(nvalid, rorder[nposc], n).astype(jnp.int32)
    ro_par = jnp.where(nvalid, parents[rorder[nposc]], n).astype(jnp.int32)
    ro_sf = jnp.where(nvalid, states[rorder[nposc]], 0).astype(jnp.float32)
    max_depth = jnp.max(depths)
    ro_tgt = jnp.where(nvalid & (d_rows[:, None] < max_depth), ro_node, n
                       ).astype(jnp.int32)

    return dict(l_idx=l_idx, l_slot=l_slot, r_idx=r_idx, r_slot=r_slot,
                h_idx=h_idx, h_slot=h_slot, h_idx_cl=h_idx_cl,
                up_tgt=up_tgt, hp_tgt=hp_tgt, hp_gidx=hp_gidx,
                up_gidx=up_gidx, ro_node=ro_node, ro_par=ro_par,
                ro_sf=ro_sf[..., None], ro_tgt=ro_tgt)


def _sds(shape, dtype=jnp.float32):
    return jax.ShapeDtypeStruct(shape, dtype)


def _mesh():
    return plsc.VectorSubcoreMesh(core_axis_name="c", subcore_axis_name="s")


def _wid():
    return lax.axis_index("s") * 2 + lax.axis_index("c")


@functools.lru_cache(maxsize=None)
def _sc_gather_h(np_rows, n):
    @functools.partial(
        pl.kernel, mesh=_mesh(), out_type=(),
        scratch_types=[pltpu.VMEM((1, 128), jnp.int32),
                       pltpu.VMEM((128, 128), jnp.float32),
                       pltpu.VMEM((128, 128), jnp.float32),
                       pltpu.SemaphoreType.DMA,
                       pltpu.SemaphoreType.DMA])
    def k(xb, plr, idx, xg, plg, idx_v, bufa, bufb, semg, semo):
        w = _wid()
        pltpu.sync_copy(idx.at[w], idx_v)
        c1 = pltpu.async_copy(xb.at[idx_v.at[0]], bufa, semg)
        c2 = pltpu.async_copy(plr.at[idx_v.at[0]], bufb, semg)
        c1.wait()
        c2.wait()
        o1 = pltpu.async_copy(bufa, xg.at[pl.ds(w * 128, 128)], semo)
        o2 = pltpu.async_copy(bufb, plg.at[pl.ds(w * 128, 128)], semo)
        o1.wait()
        o2.wait()
    return k


@functools.lru_cache(maxsize=None)
def _sc_gather_ro(np_rows):
    @functools.partial(
        pl.kernel, mesh=_mesh(), out_type=(),
        scratch_types=[pltpu.VMEM((4, 128), jnp.int32),
                       pltpu.VMEM((4, 128), jnp.int32),
                       pltpu.VMEM((512, 128), jnp.float32),
                       pltpu.VMEM((384, 128), jnp.float32),
                       pltpu.SemaphoreType.DMA,
                       pltpu.SemaphoreType.DMA])
    def k(xb, idx1, idx2, xn, xp, i1, i2, bufa, bufb, semg, semo):
        w = _wid()
        pltpu.sync_copy(idx1.at[w], i1)
        pltpu.sync_copy(idx2.at[w], i2)
        gs = [pltpu.async_copy(xb.at[i1.at[j]],
                               bufa.at[pl.ds(j * 128, 128)], semg)
              for j in range(4)]
        for c in gs:
            c.wait()
        os = [pltpu.async_copy(bufa.at[pl.ds(j * 128, 128)],
                               xn.at[pl.ds((w * 4 + j) * 128, 128)], semo)
              for j in range(4)]
        g2 = [pltpu.async_copy(xb.at[i2.at[j]],
                               bufb.at[pl.ds(j * 128, 128)], semg)
              for j in range(3)]
        for c in os + g2:
            c.wait()
        g3 = pltpu.async_copy(xb.at[i2.at[3]], bufa.at[pl.ds(0, 128)], semg)
        o2 = [pltpu.async_copy(bufb.at[pl.ds(j * 128, 128)],
                               xp.at[pl.ds((w * 4 + j) * 128, 128)], semo)
              for j in range(3)]
        g3.wait()
        o3 = pltpu.async_copy(bufa.at[pl.ds(0, 128)],
                              xp.at[pl.ds((w * 4 + 3) * 128, 128)], semo)
        for c in o2 + [o3]:
            c.wait()
    return k


@functools.lru_cache(maxsize=None)
def _sc_accum_lr(np_rows, n):
    @functools.partial(
        pl.kernel, mesh=_mesh(), out_type=(),
        scratch_types=[pltpu.VMEM_SHARED((_CAPP, 128), jnp.float32),
                       pltpu.VMEM((2, 128), jnp.int32),
                       pltpu.VMEM((2, 128), jnp.int32),
                       pltpu.VMEM((2, 128), jnp.int32),
                       pltpu.VMEM((128, 128), jnp.float32),
                       pltpu.VMEM((256, 128), jnp.float32),
                       pltpu.VMEM((256, 128), jnp.float32),
                       pltpu.SemaphoreType.DMA,
                       pltpu.SemaphoreType.DMA,
                       pltpu.SemaphoreType.DMA])
    def k(xb, pefr, lidx, lslot, ridx, rslot, hpg, upg,
          leftsum, rightsum, xd, pefu,
          acc, i1, i2, i3, zbuf, bufa, bufb, semi, semg, semo):
        c = lax.axis_index("c")
        s = lax.axis_index("s")

        def zero_row(i, carry):
            for q in range(8):
                zbuf[i, pl.ds(q * 16, 16)] = jnp.zeros((16,), jnp.float32)
            return carry
        lax.fori_loop(0, 128, zero_row, 0)

        @pl.when(c == 0)
        def _():
            cps = [pltpu.async_copy(lidx.at[s], i1, semi),
                   pltpu.async_copy(lslot.at[s], i2, semi),
                   pltpu.async_copy(hpg.at[s], i3, semi)]
            for q in range(2):
                pltpu.sync_copy(zbuf, acc.at[pl.ds(s * 256 + q * 128, 128)])
            for cp in cps:
                cp.wait()

        @pl.when(c == 1)
        def _():
            cps = [pltpu.async_copy(ridx.at[s], i1, semi),
                   pltpu.async_copy(rslot.at[s], i2, semi),
                   pltpu.async_copy(upg.at[s], i3, semi)]
            for q in range(2):
                pltpu.sync_copy(zbuf, acc.at[pl.ds(s * 256 + q * 128, 128)])
            for cp in cps:
                cp.wait()

        plsc.subcore_barrier()

        @pl.when(c == 0)
        def _():
            gs = [pltpu.async_copy(xb.at[i1.at[j]],
                                   bufa.at[pl.ds(j * 128, 128)], semg)
                  for j in range(2)]
            g2 = [pltpu.async_copy(xb.at[i3.at[j]],
                                   bufb.at[pl.ds(j * 128, 128)], semg)
                  for j in range(2)]
            for cp in gs + g2:
                cp.wait()
            for j in range(2):
                pltpu.sync_copy(bufa.at[pl.ds(j * 128, 128)],
                                acc.at[i2.at[j]], add=True)
            os = [pltpu.async_copy(bufb.at[pl.ds(j * 128, 128)],
                                   xd.at[pl.ds(s * 256 + j * 128, 128)], semo)
                  for j in range(2)]
            for cp in os:
                cp.wait()

        @pl.when(c == 1)
        def _():
            gs = [pltpu.async_copy(xb.at[i1.at[j]],
                                   bufa.at[pl.ds(j * 128, 128)], semg)
                  for j in range(2)]
            g2 = [pltpu.async_copy(pefr.at[i3.at[j]],
                                   bufb.at[pl.ds(j * 128, 128)], semg)
                  for j in range(2)]
            for cp in gs + g2:
                cp.wait()
            for j in range(2):
                pltpu.sync_copy(bufa.at[pl.ds(j * 128, 128)],
                                acc.at[i2.at[j]], add=True)
            os = [pltpu.async_copy(bufb.at[pl.ds(j * 128, 128)],
                                   pefu.at[pl.ds(s * 256 + j * 128, 128)],
                                   semo)
                  for j in range(2)]
            for cp in os:
                cp.wait()

        plsc.subcore_barrier()

        @pl.when(c == 0)
        def _():
            for q in range(2):
                sl = pl.ds(s * 256 + q * 128, 128)
                pltpu.sync_copy(acc.at[sl], leftsum.at[sl])

        @pl.when(c == 1)
        def _():
            for q in range(2):
                sl = pl.ds(s * 256 + q * 128, 128)
                pltpu.sync_copy(acc.at[sl], rightsum.at[sl])
    return k


@functools.lru_cache(maxsize=None)
def _sc_accum_h(np_rows):
    @functools.partial(
        pl.kernel, mesh=_mesh(), out_type=(),
        scratch_types=[pltpu.VMEM_SHARED((_CAPP, 128), jnp.float32),
                       pltpu.VMEM((2, 128), jnp.int32),
                       pltpu.VMEM((128, 128), jnp.float32),
                       pltpu.VMEM((256, 128), jnp.float32),
                       pltpu.SemaphoreType.DMA,
                       pltpu.SemaphoreType.DMA])
    def k(proc, hslot, mhs, acc, i1, zbuf, bufp, semi, semg):
        c = lax.axis_index("c")
        s = lax.axis_index("s")

        def zero_row(i, carry):
            for q in range(8):
                zbuf[i, pl.ds(q * 16, 16)] = jnp.zeros((16,), jnp.float32)
            return carry
        lax.fori_loop(0, 128, zero_row, 0)

        @pl.when(c == 0)
        def _():
            cp = pltpu.async_copy(hslot.at[s], i1, semi)
            for q in range(2):
                pltpu.sync_copy(zbuf, acc.at[pl.ds(s * 256 + q * 128, 128)])
            cp.wait()

        plsc.subcore_barrier()

        @pl.when(c == 0)
        def _():
            pltpu.async_copy(proc.at[pl.ds(s * 256, 256)], bufp, semg).wait()
            for j in range(2):
                pltpu.sync_copy(bufp.at[pl.ds(j * 128, 128)],
                                acc.at[i1.at[j]], add=True)

        plsc.subcore_barrier()

        @pl.when(c == 0)
        def _():
            for q in range(2):
                sl = pl.ds(s * 256 + q * 128, 128)
                pltpu.sync_copy(acc.at[sl], mhs.at[sl])
    return k


@functools.lru_cache(maxsize=None)
def _sc_scatter_pair(np_rows):
    @functools.partial(
        pl.kernel, mesh=_mesh(), out_type=(),
        scratch_types=[pltpu.VMEM((2, 128), jnp.int32),
                       pltpu.VMEM((128, 128), jnp.float32),
                       pltpu.VMEM((128, 128), jnp.float32),
                       pltpu.SemaphoreType.DMA,
                       pltpu.SemaphoreType.DMA])
    def k(up, hp, tgt, xb, idx_v, bufa, bufb, semg, semo):
        w = _wid()
        pltpu.sync_copy(tgt.at[w], idx_v)
        c1 = pltpu.async_copy(up.at[pl.ds(w * 128, 128)], bufa, semg)
        c2 = pltpu.async_copy(hp.at[pl.ds(w * 128, 128)], bufb, semg)
        c1.wait()
        c2.wait()
        s1 = pltpu.async_copy(bufa, xb.at[idx_v.at[0]], semo)
        s2 = pltpu.async_copy(bufb, xb.at[idx_v.at[1]], semo)
        s1.wait()
        s2.wait()
    return k


@functools.lru_cache(maxsize=None)
def _sc_scatter(np_rows, nk):
    @functools.partial(
        pl.kernel, mesh=_mesh(), out_type=(),
        scratch_types=[pltpu.VMEM((nk, 128), jnp.int32),
                       pltpu.VMEM((nk * 128, 128), jnp.float32),
                       pltpu.SemaphoreType.DMA,
                       pltpu.SemaphoreType.DMA])
    def k(src, tgt, xb, idx_v, bufa, semg, semo):
        w = _wid()
        pltpu.sync_copy(tgt.at[w], idx_v)
        pltpu.async_copy(src.at[pl.ds(w * nk * 128, nk * 128)], bufa,
                         semg).wait()
        ss = [pltpu.async_copy(bufa.at[pl.ds(j * 128, 128)],
                               xb.at[idx_v.at[j]], semo)
              for j in range(nk)]
        for cp in ss:
            cp.wait()
    return k


def _dot(a, b):
    return lax.dot_general(a, b, (((1,), (0,)), ((), ())),
                           precision=lax.Precision.HIGHEST,
                           preferred_element_type=jnp.float32)


def _mlp_body(n_in, *refs):
    b1 = refs[2 * n_in]
    w2 = refs[2 * n_in + 1]
    b2 = refs[2 * n_in + 2]
    out = refs[-1]
    h = b1[...].astype(jnp.float32)
    for i in range(n_in):
        xi = refs[i][...]
        wi = refs[n_in + i][...]
        if xi.shape[1] == 1:
            h = h + xi.astype(jnp.float32) * wi[...].astype(jnp.float32)
        else:
            h = h + _dot(xi, wi)
    h = jnp.maximum(h, 0.0)
    out[...] = _dot(h, w2[...]) + b2[...].astype(jnp.float32)


def _mlp(parts, b1, w2, b2, out_rows=None):
    n_in = len(parts)
    r = out_rows if out_rows is not None else parts[0][0].shape[0]
    o = w2.shape[1]
    blk = min(_BLK, r)
    grid = (pl.cdiv(r, blk),)
    in_specs = (
        [pl.BlockSpec((blk, xi.shape[1]), lambda i: (i, 0)) for xi, _ in parts]
        + [pl.BlockSpec(wi.shape, lambda i: (0, 0)) for _, wi in parts]
        + [pl.BlockSpec((1, b1.shape[1]), lambda i: (0, 0)),
           pl.BlockSpec(w2.shape, lambda i: (0, 0)),
           pl.BlockSpec((1, b2.shape[1]), lambda i: (0, 0))]
    )
    fn = pl.pallas_call(
        functools.partial(_mlp_body, n_in),
        grid=grid,
        in_specs=in_specs,
        out_specs=pl.BlockSpec((blk, o), lambda i: (i, 0)),
        out_shape=jax.ShapeDtypeStruct((r, o), jnp.float32),
    )
    args = [xi for xi, _ in parts] + [wi for _, wi in parts] + [b1, w2, b2]
    return fn(*args)


def _mlp1(x, p, out_rows=None):
    return _mlp([(x, p["W1"])], p["b1"].reshape(1, -1), p["W2"],
                p["b2"].reshape(1, -1), out_rows=out_rows)


def _pair_body(*refs):
    (ls, rs, pu, xd, mh,
     w1ml, w1mr, w1mp, b1m, w2m, b2m,
     w1lx, w1lm, b1l, w2l, b2l, oup, ohp) = refs
    h = jnp.maximum(_dot(ls[...], w1ml[...]) + _dot(rs[...], w1mr[...])
                    + _dot(pu[...], w1mp[...]) + b1m[...], 0.0)
    oup[...] = _dot(h, w2m[...]) + b2m[...]
    h2 = jnp.maximum(_dot(xd[...], w1lx[...]) + _dot(mh[...], w1lm[...])
                     + b1l[...], 0.0)
    ohp[...] = _dot(h2, w2l[...]) + b2l[...]


def _pair_call(ls, rs, pu, xd, mh, pm, plm):
    blk = 1024
    args = [ls, rs, pu, xd, mh,
            pm["W1"][:128], pm["W1"][128:256], pm["W1"][256:],
            pm["b1"].reshape(1, -1), pm["W2"], pm["b2"].reshape(1, -1),
            plm["W1"][:128], plm["W1"][128:], plm["b1"].reshape(1, -1),
            plm["W2"], plm["b2"].reshape(1, -1)]
    in_specs = [pl.BlockSpec((blk, 128), lambda i: (i, 0))] * 5 + [
        pl.BlockSpec(a.shape, lambda i: (0, 0)) for a in args[5:]]
    return pl.pallas_call(
        _pair_body, grid=(_CAPP // blk,), in_specs=in_specs,
        out_specs=(pl.BlockSpec((blk, 128), lambda i: (i, 0)),
                   pl.BlockSpec((blk, 128), lambda i: (i, 0))),
        out_shape=(_sds((_CAPP, 128)), _sds((_CAPP, 128))))(*args)


def kernel(x, parent_edge_features, parent_light_edge_features, params,
           edge_index, depths, states):
    pef_raw = parent_edge_features
    plef_raw = parent_light_edge_features
    n = x.shape[0]
    np_rows = (n // _BLK + 1) * _BLK
    parents = jnp.zeros((n,), dtype=edge_index.dtype).at[edge_index[0]].set(
        edge_index[1])
    pre = _preprocess(parents, depths, states, n)

    x_enc = _mlp1(x, params["enc"], out_rows=np_rows)
    xref = jax.new_ref(x_enc)

    pe = params["edge"]
    pm = params["merger"]
    plp = params["lep"]
    plm = params["lem"]
    pp = params["proc"]
    pef = _mlp1(pef_raw, pe)
    plef = _mlp1(plef_raw, pe)
    w1lp_x, w1lp_e = plp["W1"][:128], plp["W1"][128:]

    gh = _sc_gather_h(np_rows, n)
    gro = _sc_gather_ro(np_rows)
    acclr = _sc_accum_lr(np_rows, n)
    acch = _sc_accum_h(np_rows)
    scp = _sc_scatter_pair(np_rows)
    sc4 = _sc_scatter(np_rows, 4)

    def _refs(k, rows):
        return [jax.new_ref(jnp.zeros((rows, 128), jnp.float32))
                for _ in range(k)]

    for depth in range(MAX_DEPTH - 1, 0, -1):
        r = depth - 1
        xg_r, plg_r = _refs(2, _CAPC)
        gh(xref, plef, pre["h_idx_cl"][r].reshape(32, 1, 128), xg_r, plg_r)
        processed = _mlp([(xg_r[...], w1lp_x), (plg_r[...], w1lp_e)],
                         plp["b1"].reshape(1, -1), plp["W2"],
                         plp["b2"].reshape(1, -1))
        ls_r, rs_r, mh_r, xd_r, pu_r = _refs(5, _CAPP)
        acclr(xref, pef,
              pre["l_idx"][r].reshape(16, 2, 128),
              pre["l_slot"][r].reshape(16, 2, 128),
              pre["r_idx"][r].reshape(16, 2, 128),
              pre["r_slot"][r].reshape(16, 2, 128),
              pre["hp_gidx"][r].reshape(16, 2, 128),
              pre["up_gidx"][r].reshape(16, 2, 128),
              ls_r, rs_r, xd_r, pu_r)
        acch(processed, pre["h_slot"][r].reshape(16, 2, 128), mh_r)
        newx_up, newx_hp = _pair_call(ls_r[...], rs_r[...], pu_r[...],
                                      xd_r[...], mh_r[...], pm, plm)
        ptgt = jnp.stack([pre["up_tgt"][r].reshape(32, 128),
                          pre["hp_tgt"][r].reshape(32, 128)], axis=1)
        scp(newx_up, newx_hp, ptgt, xref)

    w1p_x, w1p_p, w1p_s = pp["W1"][:128], pp["W1"][128:256], pp["W1"][256:]
    for depth in range(1, MAX_DEPTH):
        r = depth - 1
        xn_r, xp_r = _refs(2, _CAPN)
        gro(xref,
            pre["ro_node"][r].reshape(32, 4, 128),
            pre["ro_par"][r].reshape(32, 4, 128), xn_r, xp_r)
        merged = _mlp([(xn_r[...], w1p_x), (xp_r[...], w1p_p),
                       (pre["ro_sf"][r], w1p_s)],
                      pp["b1"].reshape(1, -1), pp["W2"],
                      pp["b2"].reshape(1, -1))
        sc4(merged, pre["ro_tgt"][r].reshape(32, 4, 128), xref)

    xfin = xref[...]
    return _mlp1(xfin, params["dec"])[:n]
